# baseline (speedup 1.0000x reference)
import jax
import jax.numpy as jnp
from jax.experimental import pallas as pl
from jax.experimental.pallas import tpu as pltpu

_D_IN = 768
_N_FEATURES = 12288
_K = 32
_TILE_R = 256
_SEG = 64
_BISECT_STEPS = 16
_NEG = -1e30
_N_CHUNK = 4
_CW = _N_FEATURES // _N_CHUNK


def _sae_body(x_ref, w_ref, b_ref, o_ref):
    xv = x_ref[...]
    seg_parts = []
    for c in range(_N_CHUNK):
        sl = pl.ds(c * _CW, _CW)
        pre_c = jax.lax.dot_general(
            xv, w_ref[sl, :].astype(jnp.float32), (((1,), (1,)), ((), ())),
            preferred_element_type=jnp.float32,
        ) + b_ref[:, sl]
        o_ref[:, sl] = pre_c
        seg_parts.append(
            jnp.max(pre_c.reshape(_TILE_R, _CW // _SEG, _SEG), axis=2))

    seg_max = jnp.concatenate(seg_parts, axis=1)
    hi = jnp.max(seg_max, axis=1, keepdims=True)

    def _drop_max(_, m):
        cur = jnp.max(m, axis=1, keepdims=True)
        return jnp.where(m == cur, _NEG, m)

    m = jax.lax.fori_loop(0, _K - 1, _drop_max, seg_max)
    lo = jnp.max(m, axis=1, keepdims=True)

    def _bisect(_, carry):
        lo, hi = carry
        mid = 0.5 * (lo + hi)
        cnt = jnp.zeros((_TILE_R, 1), jnp.float32)
        for c in range(_N_CHUNK):
            sl = pl.ds(c * _CW, _CW)
            cnt += jnp.sum((o_ref[:, sl] >= mid).astype(jnp.float32), axis=1,
                           keepdims=True)
        ge = cnt >= _K
        return jnp.where(ge, mid, lo), jnp.where(ge, hi, mid)

    lo, hi = jax.lax.fori_loop(0, _BISECT_STEPS, _bisect, (lo, hi))

    for c in range(_N_CHUNK):
        sl = pl.ds(c * _CW, _CW)
        pre_c = o_ref[:, sl]
        o_ref[:, sl] = jnp.where(pre_c >= lo, jnp.maximum(pre_c, 0.0), 0.0)


def _encode(x2, W, b2):
    n = x2.shape[0]
    return pl.pallas_call(
        _sae_body,
        grid=(n // _TILE_R,),
        in_specs=[
            pl.BlockSpec((_TILE_R, _D_IN), lambda i: (i, 0)),
            pl.BlockSpec((_N_FEATURES, _D_IN), lambda i: (0, 0)),
            pl.BlockSpec((1, _N_FEATURES), lambda i: (0, 0)),
        ],
        out_specs=pl.BlockSpec((_TILE_R, _N_FEATURES), lambda i: (i, 0)),
        out_shape=jax.ShapeDtypeStruct((n, _N_FEATURES), jnp.float32),
        compiler_params=pltpu.CompilerParams(
            dimension_semantics=("arbitrary",),
        ),
    )(x2, W, b2)


def kernel(x, W, b):
    x2 = x.reshape(-1, _D_IN)
    b2 = b.reshape(1, _N_FEATURES)
    return _encode(x2, W.astype(jnp.bfloat16), b2)
